# baseline (speedup 1.0000x reference)
import functools

import jax
import jax.numpy as jnp
from jax import lax
from jax.experimental import pallas as pl
from jax.experimental.pallas import tpu as pltpu
from jax.experimental.pallas import tpu_sc as plsc

_NW = 32
_ACTIVE = 28
_CPW = 7
_CELL = 30 * 64
_CHUNK = _CPW * _CELL


def _vf(v):
    return jnp.full((16,), v, jnp.float32)


def _sc_loss(p_hbm, t_hbm, out_hbm, p_v, t_v, acc_v):
    wid = lax.axis_index("s") * 2 + lax.axis_index("c")
    acc_v[pl.ds(0, 16)] = _vf(0.0)
    acc_v[pl.ds(16, 16)] = _vf(0.0)
    acc_v[pl.ds(32, 16)] = _vf(0.0)

    @pl.when(wid < _ACTIVE)
    def _():
        base = wid * _CHUNK
        pltpu.sync_copy(p_hbm.at[pl.ds(base, _CHUNK)], p_v)
        pltpu.sync_copy(t_hbm.at[pl.ds(base, _CHUNK)], t_v)

        zero = _vf(0.0)
        one = _vf(1.0)
        half = _vf(0.5)
        inv_s = _vf(1.0 / 14.0)
        sqrt_c = jnp.full((16,), 0x1FBD1DF5, jnp.int32)
        sqrt_s = jnp.full((16,), 1, jnp.int32)

        def sqrt16(x):
            i = plsc.bitcast(x, jnp.int32)
            y = plsc.bitcast((i >> sqrt_s) + sqrt_c, jnp.float32)
            for _ in range(3):
                y = half * (y + x / y)
            return y

        loc = zero
        contain = zero
        not_contain = zero
        noo = zero
        cls = zero

        for c in range(_CPW):
            for g in range(4):

                def pch(ch):
                    return p_v[pl.ds(c * _CELL + ch * 64 + g * 16, 16)]

                def tch(ch):
                    return t_v[pl.ds(c * _CELL + ch * 64 + g * 16, 16)]

                t4 = tch(4)
                coo = jnp.where(t4 > zero, one, zero)
                nmask = one - coo

                d4 = pch(4) - t4
                d9 = pch(9) - tch(9)
                noo += nmask * (d4 * d4 + d9 * d9)

                tx = tch(0) * inv_s
                ty = tch(1) * inv_s
                tw = tch(2)
                th = tch(3)
                t_ltx = tx - half * tw
                t_lty = ty - half * th
                t_rbx = tx + half * tw
                t_rby = ty + half * th
                area2 = (t_rbx - t_ltx) * (t_rby - t_lty)

                def iou(off):
                    px = pch(off) * inv_s
                    py = pch(off + 1) * inv_s
                    pw = pch(off + 2)
                    ph = pch(off + 3)
                    p_ltx = px - half * pw
                    p_lty = py - half * ph
                    p_rbx = px + half * pw
                    p_rby = py + half * ph
                    ltx = jnp.maximum(p_ltx, t_ltx)
                    lty = jnp.maximum(p_lty, t_lty)
                    rbx = jnp.minimum(p_rbx, t_rbx)
                    rby = jnp.minimum(p_rby, t_rby)
                    whx = jnp.maximum(rbx - ltx, zero)
                    why = jnp.maximum(rby - lty, zero)
                    inter = whx * why
                    area1 = (p_rbx - p_ltx) * (p_rby - p_lty)
                    return inter / (area1 + area2 - inter)

                iou0 = iou(0)
                iou1 = iou(5)
                sel = iou1 > iou0
                max_iou = jnp.maximum(iou0, iou1)

                def pick(f, cidx):
                    return jnp.where(sel, f(5 + cidx), f(cidx))

                dx = pick(pch, 0) - pick(tch, 0)
                dy = pick(pch, 1) - pick(tch, 1)
                dw = sqrt16(pick(pch, 2)) - sqrt16(pick(tch, 2))
                dh = sqrt16(pick(pch, 3)) - sqrt16(pick(tch, 3))
                loc += coo * (dx * dx + dy * dy + dw * dw + dh * dh)

                dc = pick(pch, 4) - max_iou
                contain += coo * dc * dc
                np_c = jnp.where(sel, pch(4), pch(9))
                not_contain += coo * np_c * np_c

                csum = zero
                for ch in range(10, 30):
                    cd = pch(ch) - tch(ch)
                    csum += cd * cd
                cls += coo * csum

        loc5 = _vf(5.0) * loc
        total = (
            loc5 + _vf(2.0) * contain + not_contain + half * noo + cls
        ) * _vf(1.0 / 64.0)
        acc_v[pl.ds(0, 16)] = total
        acc_v[pl.ds(16, 16)] = loc5
        acc_v[pl.ds(32, 16)] = cls

    pltpu.sync_copy(acc_v, out_hbm.at[wid])


@jax.jit
def _run(p, t):
    mesh = plsc.VectorSubcoreMesh(core_axis_name="c", subcore_axis_name="s")
    f = functools.partial(
        pl.kernel,
        mesh=mesh,
        out_type=jax.ShapeDtypeStruct((_NW, 48), jnp.float32),
        scratch_types=[
            pltpu.VMEM((_CHUNK,), jnp.float32),
            pltpu.VMEM((_CHUNK,), jnp.float32),
            pltpu.VMEM((48,), jnp.float32),
        ],
        compiler_params=pltpu.CompilerParams(needs_layout_passes=False),
    )(_sc_loss)
    return f(p, t)


def kernel(pred_tensor, target_tensor):
    p = jnp.transpose(pred_tensor, (1, 2, 3, 0)).reshape(-1)
    t = jnp.transpose(target_tensor, (1, 2, 3, 0)).reshape(-1)
    partials = _run(p, t)
    sums = jnp.sum(partials.reshape(_NW, 3, 16), axis=(0, 2))
    return sums[0], sums[1], sums[2]


# speedup vs baseline: 5.7488x; 5.7488x over previous
import jax
import jax.numpy as jnp
from jax.experimental import pallas as pl

_S = 14.0


def _loss_kernel(p_ref, t_ref, tot_ref, loc_ref, cls_ref):
    xp = jnp.transpose(p_ref[...], (1, 0, 2))
    xt = jnp.transpose(t_ref[...], (1, 0, 2))

    def ch(arr, c):
        return arr[c]

    t4 = ch(xt, 4)
    coo = (t4 > 0.0).astype(jnp.float32)
    noo = (t4 == 0.0).astype(jnp.float32)

    d4 = ch(xp, 4) - t4
    d9 = ch(xp, 9) - ch(xt, 9)
    noo_loss = jnp.sum(noo * (d4 * d4 + d9 * d9))

    tx = ch(xt, 0) / _S
    ty = ch(xt, 1) / _S
    tw = ch(xt, 2)
    th = ch(xt, 3)
    t_ltx = tx - 0.5 * tw
    t_lty = ty - 0.5 * th
    t_rbx = tx + 0.5 * tw
    t_rby = ty + 0.5 * th
    area2 = (t_rbx - t_ltx) * (t_rby - t_lty)

    def iou(off):
        px = ch(xp, off) / _S
        py = ch(xp, off + 1) / _S
        pw = ch(xp, off + 2)
        ph = ch(xp, off + 3)
        p_ltx = px - 0.5 * pw
        p_lty = py - 0.5 * ph
        p_rbx = px + 0.5 * pw
        p_rby = py + 0.5 * ph
        ltx = jnp.maximum(p_ltx, t_ltx)
        lty = jnp.maximum(p_lty, t_lty)
        rbx = jnp.minimum(p_rbx, t_rbx)
        rby = jnp.minimum(p_rby, t_rby)
        whx = jnp.maximum(rbx - ltx, 0.0)
        why = jnp.maximum(rby - lty, 0.0)
        inter = whx * why
        area1 = (p_rbx - p_ltx) * (p_rby - p_lty)
        return inter / (area1 + area2 - inter)

    iou0 = iou(0)
    iou1 = iou(5)
    sel = iou1 > iou0
    max_iou = jnp.maximum(iou0, iou1)

    def pick(arr, c):
        return jnp.where(sel, ch(arr, 5 + c), ch(arr, c))

    rp_x = pick(xp, 0)
    rp_y = pick(xp, 1)
    rp_w = pick(xp, 2)
    rp_h = pick(xp, 3)
    rp_c = pick(xp, 4)
    rt_x = pick(xt, 0)
    rt_y = pick(xt, 1)
    rt_w = pick(xt, 2)
    rt_h = pick(xt, 3)
    np_c = jnp.where(sel, ch(xp, 4), ch(xp, 9))

    dx = rp_x - rt_x
    dy = rp_y - rt_y
    dw = jnp.sqrt(rp_w) - jnp.sqrt(rt_w)
    dh = jnp.sqrt(rp_h) - jnp.sqrt(rt_h)
    loc = jnp.sum(coo * (dx * dx + dy * dy + dw * dw + dh * dh))
    dc = rp_c - max_iou
    contain = jnp.sum(coo * dc * dc)
    not_contain = jnp.sum(coo * np_c * np_c)

    cdiff = xp[10:30] - xt[10:30]
    cls = jnp.sum(coo[None] * cdiff * cdiff)

    total = (5.0 * loc + 2.0 * contain + not_contain + 0.5 * noo_loss + cls) * (
        1.0 / 64.0
    )
    tot_ref[...] = jnp.full((1, 1), total)
    loc_ref[...] = jnp.full((1, 1), 5.0 * loc)
    cls_ref[...] = jnp.full((1, 1), cls)


def kernel(pred_tensor, target_tensor):
    p = jnp.transpose(pred_tensor, (1, 2, 3, 0)).reshape(196, 30, 64)
    t = jnp.transpose(target_tensor, (1, 2, 3, 0)).reshape(196, 30, 64)
    out_sds = jax.ShapeDtypeStruct((1, 1), jnp.float32)
    tot, loc, cls = pl.pallas_call(
        _loss_kernel,
        out_shape=(out_sds, out_sds, out_sds),
    )(p, t)
    return tot[0, 0], loc[0, 0], cls[0, 0]


# speedup vs baseline: 6.3890x; 1.1114x over previous
import jax
import jax.numpy as jnp
from jax.experimental import pallas as pl

_S = 14.0


def _loss_kernel(p_ref, t_ref, tot_ref, loc_ref, cls_ref):
    p_nat = p_ref[...]
    t_nat = t_ref[...]
    xp = jnp.transpose(p_nat[:, :10, :], (1, 0, 2))
    xt = jnp.transpose(t_nat[:, :10, :], (1, 0, 2))

    def ch(arr, c):
        return arr[c]

    t4 = ch(xt, 4)
    coo = (t4 > 0.0).astype(jnp.float32)
    noo = (t4 == 0.0).astype(jnp.float32)

    d4 = ch(xp, 4) - t4
    d9 = ch(xp, 9) - ch(xt, 9)
    noo_loss = jnp.sum(noo * (d4 * d4 + d9 * d9))

    tx = ch(xt, 0) / _S
    ty = ch(xt, 1) / _S
    tw = ch(xt, 2)
    th = ch(xt, 3)
    t_ltx = tx - 0.5 * tw
    t_lty = ty - 0.5 * th
    t_rbx = tx + 0.5 * tw
    t_rby = ty + 0.5 * th
    area2 = (t_rbx - t_ltx) * (t_rby - t_lty)

    def iou(off):
        px = ch(xp, off) / _S
        py = ch(xp, off + 1) / _S
        pw = ch(xp, off + 2)
        ph = ch(xp, off + 3)
        p_ltx = px - 0.5 * pw
        p_lty = py - 0.5 * ph
        p_rbx = px + 0.5 * pw
        p_rby = py + 0.5 * ph
        ltx = jnp.maximum(p_ltx, t_ltx)
        lty = jnp.maximum(p_lty, t_lty)
        rbx = jnp.minimum(p_rbx, t_rbx)
        rby = jnp.minimum(p_rby, t_rby)
        whx = jnp.maximum(rbx - ltx, 0.0)
        why = jnp.maximum(rby - lty, 0.0)
        inter = whx * why
        area1 = (p_rbx - p_ltx) * (p_rby - p_lty)
        return inter / (area1 + area2 - inter)

    iou0 = iou(0)
    iou1 = iou(5)
    sel = iou1 > iou0
    max_iou = jnp.maximum(iou0, iou1)

    def pick(arr, c):
        return jnp.where(sel, ch(arr, 5 + c), ch(arr, c))

    rp_x = pick(xp, 0)
    rp_y = pick(xp, 1)
    rp_w = pick(xp, 2)
    rp_h = pick(xp, 3)
    rp_c = pick(xp, 4)
    rt_x = pick(xt, 0)
    rt_y = pick(xt, 1)
    rt_w = pick(xt, 2)
    rt_h = pick(xt, 3)
    np_c = jnp.where(sel, ch(xp, 4), ch(xp, 9))

    dx = rp_x - rt_x
    dy = rp_y - rt_y
    dw = jnp.sqrt(rp_w) - jnp.sqrt(rt_w)
    dh = jnp.sqrt(rp_h) - jnp.sqrt(rt_h)
    loc = jnp.sum(coo * (dx * dx + dy * dy + dw * dw + dh * dh))
    dc = rp_c - max_iou
    contain = jnp.sum(coo * dc * dc)
    not_contain = jnp.sum(coo * np_c * np_c)

    cdiff = p_nat[:, 10:30, :] - t_nat[:, 10:30, :]
    cls = jnp.sum(coo[:, None, :] * cdiff * cdiff)

    total = (5.0 * loc + 2.0 * contain + not_contain + 0.5 * noo_loss + cls) * (
        1.0 / 64.0
    )
    tot_ref[...] = jnp.full((1, 1), total)
    loc_ref[...] = jnp.full((1, 1), 5.0 * loc)
    cls_ref[...] = jnp.full((1, 1), cls)


def kernel(pred_tensor, target_tensor):
    p = jnp.transpose(pred_tensor, (1, 2, 3, 0)).reshape(196, 30, 64)
    t = jnp.transpose(target_tensor, (1, 2, 3, 0)).reshape(196, 30, 64)
    out_sds = jax.ShapeDtypeStruct((1, 1), jnp.float32)
    tot, loc, cls = pl.pallas_call(
        _loss_kernel,
        out_shape=(out_sds, out_sds, out_sds),
    )(p, t)
    return tot[0, 0], loc[0, 0], cls[0, 0]


# speedup vs baseline: 7.4742x; 1.1699x over previous
import jax
import jax.numpy as jnp
from jax.experimental import pallas as pl

_S = 14.0


def _loss_kernel(p_ref, t_ref, tot_ref, loc_ref, cls_ref):
    p_nat = p_ref[...]
    t_nat = t_ref[...]
    xp = jnp.transpose(p_nat[:, :10, :], (1, 0, 2))
    xt = jnp.transpose(t_nat[:, :10, :], (1, 0, 2))

    def ch(arr, c):
        return arr[c]

    t4 = ch(xt, 4)
    coo = (t4 > 0.0).astype(jnp.float32)
    noo = (t4 == 0.0).astype(jnp.float32)

    d4 = ch(xp, 4) - t4
    d9 = ch(xp, 9) - ch(xt, 9)
    noo_loss = jnp.sum(noo * (d4 * d4 + d9 * d9))

    tx = ch(xt, 0) / _S
    ty = ch(xt, 1) / _S
    tw = ch(xt, 2)
    th = ch(xt, 3)
    t_ltx = tx - 0.5 * tw
    t_lty = ty - 0.5 * th
    t_rbx = tx + 0.5 * tw
    t_rby = ty + 0.5 * th
    area2 = (t_rbx - t_ltx) * (t_rby - t_lty)

    def iou(off):
        px = ch(xp, off) / _S
        py = ch(xp, off + 1) / _S
        pw = ch(xp, off + 2)
        ph = ch(xp, off + 3)
        p_ltx = px - 0.5 * pw
        p_lty = py - 0.5 * ph
        p_rbx = px + 0.5 * pw
        p_rby = py + 0.5 * ph
        ltx = jnp.maximum(p_ltx, t_ltx)
        lty = jnp.maximum(p_lty, t_lty)
        rbx = jnp.minimum(p_rbx, t_rbx)
        rby = jnp.minimum(p_rby, t_rby)
        whx = jnp.maximum(rbx - ltx, 0.0)
        why = jnp.maximum(rby - lty, 0.0)
        inter = whx * why
        area1 = (p_rbx - p_ltx) * (p_rby - p_lty)
        return inter / (area1 + area2 - inter)

    iou0 = iou(0)
    iou1 = iou(5)
    sel = iou1 > iou0
    max_iou = jnp.maximum(iou0, iou1)

    def pick(arr, c):
        return jnp.where(sel, ch(arr, 5 + c), ch(arr, c))

    rp_x = pick(xp, 0)
    rp_y = pick(xp, 1)
    rp_w = pick(xp, 2)
    rp_h = pick(xp, 3)
    rp_c = pick(xp, 4)
    rt_x = pick(xt, 0)
    rt_y = pick(xt, 1)
    rt_w = pick(xt, 2)
    rt_h = pick(xt, 3)
    np_c = jnp.where(sel, ch(xp, 4), ch(xp, 9))

    dx = rp_x - rt_x
    dy = rp_y - rt_y
    dw = jnp.sqrt(rp_w) - jnp.sqrt(rt_w)
    dh = jnp.sqrt(rp_h) - jnp.sqrt(rt_h)
    loc = jnp.sum(coo * (dx * dx + dy * dy + dw * dw + dh * dh))
    dc = rp_c - max_iou
    contain = jnp.sum(coo * dc * dc)
    not_contain = jnp.sum(coo * np_c * np_c)

    cdiff = p_nat[:, 10:30, :] - t_nat[:, 10:30, :]
    cls = jnp.sum(coo[:, None, :] * cdiff * cdiff)

    total = (5.0 * loc + 2.0 * contain + not_contain + 0.5 * noo_loss + cls) * (
        1.0 / 64.0
    )

    @pl.when(pl.program_id(0) == 0)
    def _init():
        tot_ref[...] = jnp.full((1, 1), total)
        loc_ref[...] = jnp.full((1, 1), 5.0 * loc)
        cls_ref[...] = jnp.full((1, 1), cls)

    @pl.when(pl.program_id(0) != 0)
    def _acc():
        tot_ref[...] += jnp.full((1, 1), total)
        loc_ref[...] += jnp.full((1, 1), 5.0 * loc)
        cls_ref[...] += jnp.full((1, 1), cls)


_BLK = 98


def kernel(pred_tensor, target_tensor):
    p = jnp.transpose(pred_tensor, (1, 2, 3, 0)).reshape(196, 30, 64)
    t = jnp.transpose(target_tensor, (1, 2, 3, 0)).reshape(196, 30, 64)
    out_sds = jax.ShapeDtypeStruct((1, 1), jnp.float32)
    in_spec = pl.BlockSpec((_BLK, 30, 64), lambda i: (i, 0, 0))
    out_spec = pl.BlockSpec((1, 1), lambda i: (0, 0))
    tot, loc, cls = pl.pallas_call(
        _loss_kernel,
        grid=(196 // _BLK,),
        in_specs=(in_spec, in_spec),
        out_specs=(out_spec, out_spec, out_spec),
        out_shape=(out_sds, out_sds, out_sds),
    )(p, t)
    return tot[0, 0], loc[0, 0], cls[0, 0]
